# baseline (speedup 1.0000x reference)
import functools

import jax
import jax.numpy as jnp
from jax import lax
from jax.experimental import pallas as pl
from jax.experimental.pallas import tpu as pltpu
from jax.experimental.pallas import tpu_sc as plsc

B = 16384
D = 32
LANES = 128
PACK = LANES // D
NUM_CORES = 2
NUM_SUBCORES = 16
NUM_WORKERS = NUM_CORES * NUM_SUBCORES
BPW = B // NUM_WORKERS
N_CHUNKS = 4
CROWS = BPW // N_CHUNKS
CHUNK = 16


def _sc_gather_rows(user, item, eu_view, ei_view):
    mesh = plsc.VectorSubcoreMesh(core_axis_name="c", subcore_axis_name="s")

    @functools.partial(
        pl.kernel,
        mesh=mesh,
        out_type=(
            jax.ShapeDtypeStruct((B, LANES), jnp.float32),
            jax.ShapeDtypeStruct((B, LANES), jnp.float32),
        ),
        scratch_types=[
            pltpu.VMEM((BPW,), jnp.int32),
            pltpu.VMEM((BPW,), jnp.int32),
            pltpu.VMEM((CROWS, LANES), jnp.float32),
            pltpu.VMEM((CROWS, LANES), jnp.float32),
            pltpu.SemaphoreType.DMA,
        ],
    )
    def sc_kernel(user_hbm, item_hbm, eu_hbm, ei_hbm,
                  u_out, it_out,
                  idx_u_v, idx_i_v, u_v, it_v, sem):
        wid = lax.axis_index("s") * NUM_CORES + lax.axis_index("c")
        base = wid * BPW
        pltpu.sync_copy(user_hbm.at[pl.ds(base, BPW)], idx_u_v)
        pltpu.sync_copy(item_hbm.at[pl.ds(base, BPW)], idx_i_v)

        @pl.loop(0, BPW // CHUNK)
        def _(c):
            sl = pl.ds(c * CHUNK, CHUNK)
            idx_u_v[sl] = lax.shift_right_logical(idx_u_v[sl], 2)
            idx_i_v[sl] = lax.shift_right_logical(idx_i_v[sl], 2)

        for k in range(N_CHUNKS):
            loff = k * CROWS
            copies = [
                pltpu.async_copy(
                    eu_hbm.at[idx_u_v.at[pl.ds(loff, CROWS)]], u_v, sem),
                pltpu.async_copy(
                    ei_hbm.at[idx_i_v.at[pl.ds(loff, CROWS)]], it_v, sem),
            ]
            for c in copies:
                c.wait()
            pltpu.sync_copy(u_v, u_out.at[pl.ds(base + loff, CROWS)])
            pltpu.sync_copy(it_v, it_out.at[pl.ds(base + loff, CROWS)])

    return sc_kernel(user, item, eu_view, ei_view)


def _sc_gather_bias(user, item, user_bias, item_bias):
    mesh = plsc.VectorSubcoreMesh(core_axis_name="c", subcore_axis_name="s")

    @functools.partial(
        pl.kernel,
        mesh=mesh,
        compiler_params=pltpu.CompilerParams(use_tc_tiling_on_sc=False),
        out_type=(
            jax.ShapeDtypeStruct((B,), jnp.float32),
            jax.ShapeDtypeStruct((B,), jnp.float32),
        ),
        scratch_types=[
            pltpu.VMEM((BPW,), jnp.int32),
            pltpu.VMEM((BPW,), jnp.int32),
            pltpu.VMEM((BPW,), jnp.float32),
            pltpu.VMEM((BPW,), jnp.float32),
            pltpu.SemaphoreType.DMA,
        ],
    )
    def sc_kernel(user_hbm, item_hbm, ubt_hbm, ibt_hbm,
                  ub_out, ib_out,
                  idx_u_v, idx_i_v, ub_v, ib_v, sem):
        wid = lax.axis_index("s") * NUM_CORES + lax.axis_index("c")
        base = wid * BPW
        pltpu.sync_copy(user_hbm.at[pl.ds(base, BPW)], idx_u_v)
        pltpu.sync_copy(item_hbm.at[pl.ds(base, BPW)], idx_i_v)
        copies = [
            pltpu.async_copy(ubt_hbm.at[idx_u_v], ub_v, sem),
            pltpu.async_copy(ibt_hbm.at[idx_i_v], ib_v, sem),
        ]
        for c in copies:
            c.wait()
        pltpu.sync_copy(ub_v, ub_out.at[pl.ds(base, BPW)])
        pltpu.sync_copy(ib_v, ib_out.at[pl.ds(base, BPW)])

    return sc_kernel(user, item, user_bias, item_bias)


def _finish_body(avg_ref, uc_ref, ic_ref, u128_ref, it128_ref, ub_ref, ib_ref,
                 pred_ref, u_ref, it_ref):
    def extract(cls, rows_ref):
        parts = [rows_ref[:, i * D:(i + 1) * D] for i in range(PACK)]
        out = parts[PACK - 1]
        for i in range(PACK - 2, -1, -1):
            out = jnp.where(cls == i, parts[i], out)
        return out

    cls_u = lax.bitwise_and(uc_ref[...], 3)
    cls_i = lax.bitwise_and(ic_ref[...], 3)
    u = extract(cls_u, u128_ref)
    it = extract(cls_i, it128_ref)
    u_ref[...] = u
    it_ref[...] = it
    s = jnp.sum(u * it, axis=1, keepdims=True)
    pred_ref[...] = s + ub_ref[...] + ib_ref[...] + avg_ref[0, 0]


def _tc_finish(average, user_c, item_c, u128, it128, ub_c, ib_c):
    grid = 4
    rows = B // grid
    return pl.pallas_call(
        _finish_body,
        grid=(grid,),
        in_specs=[
            pl.BlockSpec((1, 1), lambda i: (0, 0)),
            pl.BlockSpec((rows, 1), lambda i: (i, 0)),
            pl.BlockSpec((rows, 1), lambda i: (i, 0)),
            pl.BlockSpec((rows, LANES), lambda i: (i, 0)),
            pl.BlockSpec((rows, LANES), lambda i: (i, 0)),
            pl.BlockSpec((rows, 1), lambda i: (i, 0)),
            pl.BlockSpec((rows, 1), lambda i: (i, 0)),
        ],
        out_specs=[
            pl.BlockSpec((rows, 1), lambda i: (i, 0)),
            pl.BlockSpec((rows, D), lambda i: (i, 0)),
            pl.BlockSpec((rows, D), lambda i: (i, 0)),
        ],
        out_shape=(
            jax.ShapeDtypeStruct((B, 1), jnp.float32),
            jax.ShapeDtypeStruct((B, D), jnp.float32),
            jax.ShapeDtypeStruct((B, D), jnp.float32),
        ),
    )(average.reshape(1, 1), user_c, item_c, u128, it128, ub_c, ib_c)


def kernel(user, item, average, embed_user, embed_item, user_bias, item_bias):
    n_user = embed_user.shape[0]
    n_item = embed_item.shape[0]
    eu_view = embed_user.reshape(n_user // PACK, LANES)
    ei_view = embed_item.reshape(n_item // PACK, LANES)
    u128, it128 = _sc_gather_rows(user, item, eu_view, ei_view)
    ub, ib = _sc_gather_bias(user, item, user_bias, item_bias)
    predict, u, it = _tc_finish(
        average,
        user.reshape(B, 1),
        item.reshape(B, 1),
        u128,
        it128,
        ub.reshape(B, 1),
        ib.reshape(B, 1),
    )
    return (predict.reshape(B), u, it)


# speedup vs baseline: 2.4648x; 2.4648x over previous
import functools

import jax
import jax.numpy as jnp
from jax import lax
from jax.experimental import pallas as pl
from jax.experimental.pallas import tpu as pltpu
from jax.experimental.pallas import tpu_sc as plsc

B = 16384
D = 32
TROWS = 8
NUM_CORES = 2
NUM_SUBCORES = 16
NUM_WORKERS = NUM_CORES * NUM_SUBCORES
BPW = B // NUM_WORKERS
N_CHUNKS = 4
CROWS = BPW // N_CHUNKS
VEC = 16


def _sc_gather_rows(user, item, eu3, ei3):
    mesh = plsc.VectorSubcoreMesh(core_axis_name="c", subcore_axis_name="s")

    @functools.partial(
        pl.kernel,
        mesh=mesh,
        out_type=(
            jax.ShapeDtypeStruct((B, D), jnp.float32),
            jax.ShapeDtypeStruct((B, D), jnp.float32),
        ),
        scratch_types=[
            pltpu.VMEM((BPW,), jnp.int32),
            pltpu.VMEM((BPW,), jnp.int32),
            pltpu.VMEM((CROWS, D), jnp.float32),
            pltpu.VMEM((CROWS, D), jnp.float32),
            pltpu.SemaphoreType.DMA,
            pltpu.SemaphoreType.DMA,
        ],
    )
    def sc_kernel(user_hbm, item_hbm, eu_hbm, ei_hbm,
                  u_out, it_out,
                  idx_u_v, idx_i_v, u_v, it_v,
                  sem_u, sem_i):
        wid = lax.axis_index("s") * NUM_CORES + lax.axis_index("c")
        base = wid * BPW
        pltpu.sync_copy(user_hbm.at[pl.ds(base, BPW)], idx_u_v)
        pltpu.sync_copy(item_hbm.at[pl.ds(base, BPW)], idx_i_v)

        @pl.loop(0, N_CHUNKS)
        def _(k):
            loff = k * CROWS

            @pl.loop(0, CROWS // VEC)
            def _(c):
                vu = idx_u_v[pl.ds(loff + c * VEC, VEC)]
                vi = idx_i_v[pl.ds(loff + c * VEC, VEC)]
                for e in range(VEC):
                    iu = vu[e]
                    ii = vi[e]
                    j = c * VEC + e
                    pltpu.make_async_copy(
                        eu_hbm.at[lax.shift_right_logical(iu, 3),
                                  lax.bitwise_and(iu, TROWS - 1)],
                        u_v.at[j], sem_u).start()
                    pltpu.make_async_copy(
                        ei_hbm.at[lax.shift_right_logical(ii, 3),
                                  lax.bitwise_and(ii, TROWS - 1)],
                        it_v.at[j], sem_i).start()

            pltpu.make_async_copy(
                eu_hbm.at[0], u_v, sem_u).wait()
            pltpu.make_async_copy(
                ei_hbm.at[0], it_v, sem_i).wait()

            pltpu.sync_copy(u_v, u_out.at[pl.ds(base + loff, CROWS)])
            pltpu.sync_copy(it_v, it_out.at[pl.ds(base + loff, CROWS)])

    return sc_kernel(user, item, eu3, ei3)


def _sc_gather_bias(user, item, user_bias, item_bias):
    mesh = plsc.VectorSubcoreMesh(core_axis_name="c", subcore_axis_name="s")

    @functools.partial(
        pl.kernel,
        mesh=mesh,
        compiler_params=pltpu.CompilerParams(use_tc_tiling_on_sc=False),
        out_type=(
            jax.ShapeDtypeStruct((B,), jnp.float32),
            jax.ShapeDtypeStruct((B,), jnp.float32),
        ),
        scratch_types=[
            pltpu.VMEM((BPW,), jnp.int32),
            pltpu.VMEM((BPW,), jnp.int32),
            pltpu.VMEM((BPW,), jnp.float32),
            pltpu.VMEM((BPW,), jnp.float32),
            pltpu.SemaphoreType.DMA,
        ],
    )
    def sc_kernel(user_hbm, item_hbm, ubt_hbm, ibt_hbm,
                  ub_out, ib_out,
                  idx_u_v, idx_i_v, ub_v, ib_v, sem):
        wid = lax.axis_index("s") * NUM_CORES + lax.axis_index("c")
        base = wid * BPW
        pltpu.sync_copy(user_hbm.at[pl.ds(base, BPW)], idx_u_v)
        pltpu.sync_copy(item_hbm.at[pl.ds(base, BPW)], idx_i_v)
        copies = [
            pltpu.async_copy(ubt_hbm.at[idx_u_v], ub_v, sem),
            pltpu.async_copy(ibt_hbm.at[idx_i_v], ib_v, sem),
        ]
        for c in copies:
            c.wait()
        pltpu.sync_copy(ub_v, ub_out.at[pl.ds(base, BPW)])
        pltpu.sync_copy(ib_v, ib_out.at[pl.ds(base, BPW)])

    return sc_kernel(user, item, user_bias, item_bias)


def _finish_body(avg_ref, u_ref, it_ref, ub_ref, ib_ref, pred_ref):
    s = jnp.sum(u_ref[...] * it_ref[...], axis=1, keepdims=True)
    pred_ref[...] = s + ub_ref[...] + ib_ref[...] + avg_ref[0, 0]


def _tc_finish(average, u, it, ub_c, ib_c):
    grid = 4
    rows = B // grid
    return pl.pallas_call(
        _finish_body,
        grid=(grid,),
        in_specs=[
            pl.BlockSpec((1, 1), lambda i: (0, 0)),
            pl.BlockSpec((rows, D), lambda i: (i, 0)),
            pl.BlockSpec((rows, D), lambda i: (i, 0)),
            pl.BlockSpec((rows, 1), lambda i: (i, 0)),
            pl.BlockSpec((rows, 1), lambda i: (i, 0)),
        ],
        out_specs=pl.BlockSpec((rows, 1), lambda i: (i, 0)),
        out_shape=jax.ShapeDtypeStruct((B, 1), jnp.float32),
    )(average.reshape(1, 1), u, it, ub_c, ib_c)


def kernel(user, item, average, embed_user, embed_item, user_bias, item_bias):
    n_user = embed_user.shape[0]
    n_item = embed_item.shape[0]
    eu3 = embed_user.reshape(n_user // TROWS, TROWS, D)
    ei3 = embed_item.reshape(n_item // TROWS, TROWS, D)
    u, it = _sc_gather_rows(user, item, eu3, ei3)
    ub, ib = _sc_gather_bias(user, item, user_bias, item_bias)
    predict = _tc_finish(average, u, it, ub.reshape(B, 1), ib.reshape(B, 1))
    return (predict.reshape(B), u, it)
